# Initial kernel scaffold; baseline (speedup 1.0000x reference)
import jax
import jax.numpy as jnp
from jax.experimental import pallas as pl


def kernel(h, edge_index, Wq, Wk, Wv, Wo, gamma, beta):
    raise NotImplementedError("write your pallas kernel here")


# speedup vs baseline: 1.0430x; 1.0430x over previous
import functools

import jax
import jax.numpy as jnp
from jax.experimental import pallas as pl

N_NODES = 10000
N_EDGES = 160000
EMBED = 256
H = 8
DK = 64
DV = 64

_ROW_BLK = 1000


def _qkv_kernel(h_ref, w_ref, o_ref):
    o_ref[...] = jax.lax.dot_general(
        h_ref[...], w_ref[...], (((1,), (0,)), ((), ())),
        preferred_element_type=jnp.float32,
        precision=jax.lax.Precision.HIGHEST,
    )


def _out_ln_kernel(agg_ref, h_ref, wo_ref, g_ref, b_ref, o_ref):
    out = jax.lax.dot_general(
        agg_ref[...], wo_ref[...], (((1,), (0,)), ((), ())),
        preferred_element_type=jnp.float32,
        precision=jax.lax.Precision.HIGHEST,
    )
    y = out + h_ref[...]
    mu = jnp.mean(y, axis=-1, keepdims=True)
    var = jnp.mean((y - mu) ** 2, axis=-1, keepdims=True)
    ynorm = (y - mu) * jax.lax.rsqrt(var + 1e-5)
    o_ref[...] = ynorm * g_ref[...] + b_ref[...]


def kernel(h, edge_index, Wq, Wk, Wv, Wo, gamma, beta):
    N = h.shape[0]
    src = edge_index[0].astype(jnp.int32)
    dst = edge_index[1].astype(jnp.int32)

    Wcat = jnp.concatenate([Wq, Wk, Wv], axis=1)
    nblk = N // _ROW_BLK
    qkv = pl.pallas_call(
        _qkv_kernel,
        grid=(nblk,),
        in_specs=[
            pl.BlockSpec((_ROW_BLK, EMBED), lambda i: (i, 0)),
            pl.BlockSpec((EMBED, 3 * H * DK), lambda i: (0, 0)),
        ],
        out_specs=pl.BlockSpec((_ROW_BLK, 3 * H * DK), lambda i: (i, 0)),
        out_shape=jax.ShapeDtypeStruct((N, 3 * H * DK), jnp.float32),
    )(h, Wcat)
    q = qkv[:, : H * DK].reshape(N, H, DK)
    k = qkv[:, H * DK : 2 * H * DK].reshape(N, H, DK)
    v = qkv[:, 2 * H * DK :].reshape(N, H, DV)

    qd = q[dst]
    ks_ = k[src]
    scores = jnp.sum(qd * ks_, axis=-1) / jnp.sqrt(jnp.float32(DK))
    ex = jnp.exp(scores)
    denom = jax.ops.segment_sum(ex, dst, num_segments=N)
    u = jax.ops.segment_sum(v[src] * ex[..., None], dst, num_segments=N)
    agg = jnp.where(denom[..., None] > 0, u / denom[..., None], 0.0)
    agg = agg.reshape(N, H * DV)

    out = pl.pallas_call(
        _out_ln_kernel,
        grid=(nblk,),
        in_specs=[
            pl.BlockSpec((_ROW_BLK, H * DV), lambda i: (i, 0)),
            pl.BlockSpec((_ROW_BLK, EMBED), lambda i: (i, 0)),
            pl.BlockSpec((H * DV, EMBED), lambda i: (0, 0)),
            pl.BlockSpec((1, EMBED), lambda i: (0, 0)),
            pl.BlockSpec((1, EMBED), lambda i: (0, 0)),
        ],
        out_specs=pl.BlockSpec((_ROW_BLK, EMBED), lambda i: (i, 0)),
        out_shape=jax.ShapeDtypeStruct((N, EMBED), jnp.float32),
    )(agg, h, Wo, gamma.reshape(1, EMBED), beta.reshape(1, EMBED))
    return out


# speedup vs baseline: 8.2408x; 7.9010x over previous
import dataclasses

import jax
import jax.numpy as jnp
from jax import lax
from jax.experimental import pallas as pl
from jax.experimental.pallas import tpu as pltpu
from jax.experimental.pallas import tpu_sc as plsc

N_NODES = 10000
N_EDGES = 160000
EMBED = 256
H = 8
DK = 64
DV = 64

CHUNK = 64
N_CHUNKS = N_EDGES // CHUNK
NW = 32
CPW = (N_CHUNKS + NW - 1) // NW
NPASS = 4
PCOLS = 2 * DK
UROW = PCOLS + 16
NPAD = 10240
RSUB = NPAD // 16
ZROWS = 128

_ROW_BLK = 1000


def _qkv_kernel(h_ref, wq_ref, wk_ref, wv_ref, oq_ref, ok_ref, ov_ref):
    hb = h_ref[...]
    dn = (((1,), (0,)), ((), ()))
    oq_ref[...] = lax.dot_general(hb, wq_ref[...], dn,
                                  preferred_element_type=jnp.float32,
                                  precision=lax.Precision.HIGHEST)
    ok_ref[...] = lax.dot_general(hb, wk_ref[...], dn,
                                  preferred_element_type=jnp.float32,
                                  precision=lax.Precision.HIGHEST)
    ov_ref[...] = lax.dot_general(hb, wv_ref[...], dn,
                                  preferred_element_type=jnp.float32,
                                  precision=lax.Precision.HIGHEST)


def _edge_kernel(q4, k4, v4, src_hbm, dst_hbm, zeros_hbm, u_hbm,
                 sidx, didx, qgidx, sgidx, qrows, krows, vrows, msg,
                 ushared, sem_q, sem_k, sem_v):
    core = lax.axis_index("c")
    sid = lax.axis_index("s")
    wid = sid * 2 + core
    lanes = lax.iota(jnp.int32, 16)

    for p in range(NPASS):
        for b in range(RSUB // ZROWS):
            pltpu.sync_copy(zeros_hbm,
                            ushared.at[pl.ds(sid * RSUB + b * ZROWS, ZROWS)])
        plsc.subcore_barrier()

        @pl.loop(0, CPW)
        def _chunks(r):
            c = wid + r * NW

            @pl.when(c < N_CHUNKS)
            def _():
                pltpu.sync_copy(src_hbm.at[pl.ds(c * CHUNK, CHUNK)], sidx)
                pltpu.sync_copy(dst_hbm.at[pl.ds(c * CHUNK, CHUNK)], didx)
                for b in range(CHUNK // 16):
                    sl = pl.ds(b * 16, 16)
                    qgidx[sl] = didx[sl] * NPASS + p
                    sgidx[sl] = sidx[sl] * NPASS + p
                cq = pltpu.async_copy(q4.at[qgidx], qrows, sem_q)
                ck = pltpu.async_copy(k4.at[sgidx], krows, sem_k)
                cv = pltpu.async_copy(v4.at[sgidx], vrows, sem_v)
                cq.wait()
                ck.wait()
                cv.wait()

                @pl.loop(0, CHUNK)
                def _edges(e):
                    pr = [qrows[e, pl.ds(t * 16, 16)] * krows[e, pl.ds(t * 16, 16)]
                          for t in range(8)]
                    s_a = jnp.sum((pr[0] + pr[1]) + (pr[2] + pr[3])) * 0.125
                    s_b = jnp.sum((pr[4] + pr[5]) + (pr[6] + pr[7])) * 0.125
                    exa = jnp.exp(jnp.full((16,), s_a, jnp.float32))
                    exb = jnp.exp(jnp.full((16,), s_b, jnp.float32))
                    for t in range(8):
                        sl = pl.ds(t * 16, 16)
                        msg[e, sl] = vrows[e, sl] * (exa if t < 4 else exb)
                    msg[e, pl.ds(PCOLS, 16)] = jnp.where(lanes < 8, exa, exb)

                pltpu.sync_copy(msg, ushared.at[didx], add=True)

        plsc.subcore_barrier()

        @pl.when(sid < 15)
        def _wb_full():
            pltpu.sync_copy(ushared.at[pl.ds(sid * RSUB, RSUB)],
                            u_hbm.at[core, p, pl.ds(sid * RSUB, RSUB)])

        @pl.when(sid == 15)
        def _wb_tail():
            pltpu.sync_copy(ushared.at[pl.ds(15 * RSUB, N_NODES - 15 * RSUB)],
                            u_hbm.at[core, p, pl.ds(15 * RSUB, N_NODES - 15 * RSUB)])

        plsc.subcore_barrier()


def _out_ln_kernel(u_ref, h_ref, wo_ref, g_ref, b_ref, o_ref):
    u = u_ref[...]
    us = u[0] + u[1]
    parts = []
    for p in range(NPASS):
        for hh in range(2):
            num = us[p, :, hh * DK:(hh + 1) * DK]
            den = us[p, :, PCOLS + hh * 8:PCOLS + hh * 8 + 1]
            parts.append(jnp.where(den > 0, num / den, 0.0))
    agg = jnp.concatenate(parts, axis=1)
    out = lax.dot_general(agg, wo_ref[...], (((1,), (0,)), ((), ())),
                          preferred_element_type=jnp.float32,
                          precision=lax.Precision.HIGHEST)
    y = out + h_ref[...]
    mu = jnp.mean(y, axis=-1, keepdims=True)
    var = jnp.mean((y - mu) ** 2, axis=-1, keepdims=True)
    ynorm = (y - mu) * lax.rsqrt(var + 1e-5)
    o_ref[...] = ynorm * g_ref[...] + b_ref[...]


def kernel(h, edge_index, Wq, Wk, Wv, Wo, gamma, beta):
    N = h.shape[0]
    src = edge_index[0].astype(jnp.int32)
    dst = edge_index[1].astype(jnp.int32)

    nblk = N // _ROW_BLK
    q, k, v = pl.pallas_call(
        _qkv_kernel,
        grid=(nblk,),
        in_specs=[
            pl.BlockSpec((_ROW_BLK, EMBED), lambda i: (i, 0)),
            pl.BlockSpec((EMBED, H * DK), lambda i: (0, 0)),
            pl.BlockSpec((EMBED, H * DK), lambda i: (0, 0)),
            pl.BlockSpec((EMBED, H * DV), lambda i: (0, 0)),
        ],
        out_specs=[
            pl.BlockSpec((_ROW_BLK, H * DK), lambda i: (i, 0)),
            pl.BlockSpec((_ROW_BLK, H * DK), lambda i: (i, 0)),
            pl.BlockSpec((_ROW_BLK, H * DV), lambda i: (i, 0)),
        ],
        out_shape=[
            jax.ShapeDtypeStruct((N, H * DK), jnp.float32),
            jax.ShapeDtypeStruct((N, H * DK), jnp.float32),
            jax.ShapeDtypeStruct((N, H * DV), jnp.float32),
        ],
    )(h, Wq, Wk, Wv)

    q4 = q.reshape(N * NPASS, PCOLS)
    k4 = k.reshape(N * NPASS, PCOLS)
    v4 = v.reshape(N * NPASS, PCOLS)

    mesh = plsc.VectorSubcoreMesh(core_axis_name="c", subcore_axis_name="s")
    cp = pltpu.CompilerParams()
    fields = pltpu.CompilerParams.__dataclass_fields__
    if "needs_layout_passes" in fields:
        cp = dataclasses.replace(cp, needs_layout_passes=False)
    if "use_tc_tiling_on_sc" in fields:
        cp = dataclasses.replace(cp, use_tc_tiling_on_sc=False)
    edge_k = pl.kernel(
        _edge_kernel,
        out_type=jax.ShapeDtypeStruct((2, NPASS, N_NODES, UROW), jnp.float32),
        mesh=mesh,
        compiler_params=cp,
        scratch_types=[
            pltpu.VMEM((CHUNK,), jnp.int32),
            pltpu.VMEM((CHUNK,), jnp.int32),
            pltpu.VMEM((CHUNK,), jnp.int32),
            pltpu.VMEM((CHUNK,), jnp.int32),
            pltpu.VMEM((CHUNK, PCOLS), jnp.float32),
            pltpu.VMEM((CHUNK, PCOLS), jnp.float32),
            pltpu.VMEM((CHUNK, PCOLS), jnp.float32),
            pltpu.VMEM((CHUNK, UROW), jnp.float32),
            pltpu.VMEM_SHARED((NPAD, UROW), jnp.float32),
            pltpu.SemaphoreType.DMA,
            pltpu.SemaphoreType.DMA,
            pltpu.SemaphoreType.DMA,
        ],
    )
    zeros = jnp.zeros((ZROWS, UROW), jnp.float32)
    u = edge_k(q4, k4, v4, src, dst, zeros)

    out = pl.pallas_call(
        _out_ln_kernel,
        grid=(nblk,),
        in_specs=[
            pl.BlockSpec((2, NPASS, _ROW_BLK, UROW), lambda i: (0, 0, i, 0)),
            pl.BlockSpec((_ROW_BLK, EMBED), lambda i: (i, 0)),
            pl.BlockSpec((H * DV, EMBED), lambda i: (0, 0)),
            pl.BlockSpec((1, EMBED), lambda i: (0, 0)),
            pl.BlockSpec((1, EMBED), lambda i: (0, 0)),
        ],
        out_specs=pl.BlockSpec((_ROW_BLK, EMBED), lambda i: (i, 0)),
        out_shape=jax.ShapeDtypeStruct((N, EMBED), jnp.float32),
    )(u, h, Wo, gamma.reshape(1, EMBED), beta.reshape(1, EMBED))
    return out
